# baseline (speedup 1.0000x reference)
import functools

import jax
import jax.numpy as jnp
from jax import lax
from jax.experimental import pallas as pl
from jax.experimental.pallas import tpu as pltpu
from jax.experimental.pallas import tpu_sc as plsc

_B, _M, _D = 1024, 256, 128
_B_SC = 256
_B_TC = _B - _B_SC
_NC, _NS = 2, 16
_NW = _NC * _NS
_BPW = _B_SC // _NW
_MD = _M * _D


def _tc_body(scal_ref, h_ref, w_ref, emb_ref, ls_ref, ne_ref, logits_ref, mask_ref):
    ds = scal_ref[0, 0]
    ts = scal_ref[0, 1]
    h2 = jnp.dot(h_ref[:], w_ref[:], preferred_element_type=jnp.float32)
    h2b = h2.astype(jnp.bfloat16)
    rows = []
    for b in range(h_ref.shape[0]):
        e_b = emb_ref[b].astype(jnp.bfloat16)
        rows.append(lax.dot_general(
            h2b[b:b + 1, :], e_b,
            dimension_numbers=(((1,), (1,)), ((), ())),
            preferred_element_type=jnp.float32))
    bl = jnp.concatenate(rows, axis=0)
    dist = jnp.exp(ds * (ls_ref[:].astype(jnp.float32) - ts))
    logits = bl + dist
    bb, m = logits.shape
    iota = lax.broadcasted_iota(jnp.int32, (bb, m), 1)
    mask = iota < ne_ref[:]
    logits_ref[:] = jnp.where(mask, logits, jnp.float32(1e-34))
    mask_ref[:] = mask.astype(jnp.int32)


def _h2_body(h_ref, w_ref, out_ref):
    out_ref[:] = jnp.dot(h_ref[:], w_ref[:], preferred_element_type=jnp.float32)


def _sc_kernel(emb_hbm, h2_hbm, ls_hbm, ne_hbm, cst_hbm, out_hbm, mask_hbm,
               embbuf, h2v, lsv, nev, cstv, orow, mrow, tbuf, sem0, sem1):
    wid = lax.axis_index("s") * _NC + lax.axis_index("c")
    base = wid * _BPW
    emb_base = _B_TC + base
    pltpu.sync_copy(h2_hbm.at[pl.ds(base * _D, _BPW * _D)], h2v)
    pltpu.sync_copy(ls_hbm.at[pl.ds(base * _M, _BPW * _M)], lsv)
    pltpu.sync_copy(ne_hbm, nev)
    pltpu.sync_copy(cst_hbm, cstv)
    iota = lax.broadcasted_iota(jnp.int32, (16,), 0)
    dsv = cstv[pl.ds(0, 16)]
    tsv = cstv[pl.ds(16, 16)]
    sems = [sem0, sem1]
    handles = [None, None]
    handles[0] = pltpu.async_copy(
        emb_hbm.at[emb_base], embbuf.at[pl.ds(0, _MD)], sem0)
    for bl in range(_BPW):
        buf = bl % 2
        handles[buf].wait()
        if bl + 1 < _BPW:
            nbuf = (bl + 1) % 2
            handles[nbuf] = pltpu.async_copy(
                emb_hbm.at[emb_base + bl + 1],
                embbuf.at[pl.ds(nbuf * _MD, _MD)], sems[nbuf])
        bufbase = buf * _MD
        h2k = [h2v[pl.ds(bl * _D + k * 16, 16)] for k in range(8)]
        nesp = plsc.load_gather(nev, [jnp.full((16,), base + bl, jnp.int32)])

        def group_body(mg, carry):
            accs = []
            for j in range(16):
                moff = bufbase + (mg * 16 + j) * _D
                p = [embbuf[pl.ds(moff + k * 16, 16)] * h2k[k]
                     for k in range(8)]
                q = [p[0] + p[1], p[2] + p[3], p[4] + p[5], p[6] + p[7]]
                accs.append((q[0] + q[1]) + (q[2] + q[3]))
            for j in range(16):
                tbuf[pl.ds(j * 16, 16)] = accs[j]
            cols = [plsc.load_gather(tbuf, [iota * 16 + c]) for c in range(16)]
            while len(cols) > 1:
                cols = [cols[i] + cols[i + 1] for i in range(0, len(cols), 2)]
            tot = cols[0]
            lsx = lsv[pl.ds(bl * _M + mg * 16, 16)]
            dist = jnp.exp(dsv * (lsx.astype(jnp.float32) - tsv))
            logit = tot + dist
            mvec = iota + mg * 16
            mask = mvec < nesp
            orow[pl.ds(mg * 16, 16)] = jnp.where(mask, logit, jnp.float32(1e-34))
            mrow[pl.ds(mg * 16, 16)] = mask.astype(jnp.int32)
            return carry

        lax.fori_loop(0, _M // 16, group_body, 0)
        pltpu.sync_copy(orow, out_hbm.at[base + bl])
        pltpu.sync_copy(mrow, mask_hbm.at[base + bl])


def _run_sc(emb_sc, h2_sc, ls_sc, ne_sc, consts, interpret=False):
    mesh = plsc.VectorSubcoreMesh(core_axis_name="c", subcore_axis_name="s")
    fn = functools.partial(
        pl.kernel, mesh=mesh,
        out_type=[jax.ShapeDtypeStruct((_B_SC, _M), jnp.float32),
                  jax.ShapeDtypeStruct((_B_SC, _M), jnp.int32)],
        scratch_types=[
            pltpu.VMEM((2 * _MD,), jnp.float32),
            pltpu.VMEM((_BPW * _D,), jnp.float32),
            pltpu.VMEM((_BPW * _M,), jnp.int32),
            pltpu.VMEM((_B_SC,), jnp.int32),
            pltpu.VMEM((32,), jnp.float32),
            pltpu.VMEM((_M,), jnp.float32),
            pltpu.VMEM((_M,), jnp.int32),
            pltpu.VMEM((_M,), jnp.float32),
            pltpu.SemaphoreType.DMA,
            pltpu.SemaphoreType.DMA,
        ],
        compiler_params=pltpu.CompilerParams(needs_layout_passes=False),
        interpret=interpret)(_sc_kernel)
    return fn(emb_sc, h2_sc, ls_sc, ne_sc, consts)


@functools.partial(jax.jit, static_argnames=("interpret",))
def _run(hidden, embeddings, W_proj, distance_scalar, last_seen,
         num_embeddings, timestep, interpret=False):
    B, M, D = embeddings.shape
    Bb = 64
    ds = distance_scalar.astype(jnp.float32)
    ts = jnp.asarray(timestep, jnp.float32)
    scal = jnp.stack([ds, ts]).reshape(1, 2)
    ne = num_embeddings.astype(jnp.int32)
    ls = last_seen.astype(jnp.int32)

    h2_sc = pl.pallas_call(
        _h2_body,
        out_shape=jax.ShapeDtypeStruct((_B_SC, D), jnp.float32),
        interpret=interpret,
    )(hidden[_B_TC:], W_proj)
    consts = jnp.concatenate([jnp.full((16,), ds, jnp.float32),
                              jnp.full((16,), ts, jnp.float32)])
    sc_logits, sc_mask = _run_sc(
        embeddings.reshape(B, _MD),
        h2_sc.reshape(_B_SC * _D),
        ls[_B_TC:].reshape(_B_SC * _M),
        ne[_B_TC:],
        consts, interpret=interpret)

    grid = (_B_TC // Bb,)
    tc_logits, tc_mask = pl.pallas_call(
        _tc_body,
        grid=grid,
        in_specs=[
            pl.BlockSpec((1, 2), lambda i: (0, 0)),
            pl.BlockSpec((Bb, D), lambda i: (i, 0)),
            pl.BlockSpec((D, D), lambda i: (0, 0)),
            pl.BlockSpec((Bb, M, D), lambda i: (i, 0, 0)),
            pl.BlockSpec((Bb, M), lambda i: (i, 0)),
            pl.BlockSpec((Bb, 1), lambda i: (i, 0)),
        ],
        out_specs=[
            pl.BlockSpec((Bb, M), lambda i: (i, 0)),
            pl.BlockSpec((Bb, M), lambda i: (i, 0)),
        ],
        out_shape=[
            jax.ShapeDtypeStruct((_B_TC, M), jnp.float32),
            jax.ShapeDtypeStruct((_B_TC, M), jnp.int32),
        ],
        compiler_params=pltpu.CompilerParams(
            dimension_semantics=("parallel",),
            vmem_limit_bytes=100 * 1024 * 1024),
        interpret=interpret,
    )(scal, hidden[:_B_TC], W_proj, embeddings[:_B_TC], ls[:_B_TC],
      ne[:_B_TC].reshape(_B_TC, 1))

    logits = jnp.concatenate([tc_logits, sc_logits], axis=0)
    mask = jnp.concatenate([tc_mask, sc_mask], axis=0).astype(jnp.bool_)
    return logits, mask


def kernel(hidden, embeddings, W_proj, distance_scalar, last_seen,
           num_embeddings, timestep):
    return _run(hidden, embeddings, W_proj, distance_scalar, last_seen,
                num_embeddings, timestep)


# speedup vs baseline: 1.6484x; 1.6484x over previous
import functools

import jax
import jax.numpy as jnp
from jax import lax
from jax.experimental import pallas as pl
from jax.experimental.pallas import tpu as pltpu
from jax.experimental.pallas import tpu_sc as plsc

_B, _M, _D = 1024, 256, 128
_B_SC = 256
_B_TC = _B - _B_SC
_NC, _NS = 2, 16
_NW = _NC * _NS
_BPW = _B_SC // _NW
_MD = _M * _D


def _tc_body(scal_ref, h_ref, w_ref, emb_ref, ls_ref, ne_ref, logits_ref, mask_ref):
    ds = scal_ref[0, 0]
    ts = scal_ref[0, 1]
    h2 = jnp.dot(h_ref[:], w_ref[:], preferred_element_type=jnp.float32)
    h2b = h2.astype(jnp.bfloat16)
    rows = []
    for b in range(h_ref.shape[0]):
        e_b = emb_ref[b].astype(jnp.bfloat16)
        rows.append(lax.dot_general(
            h2b[b:b + 1, :], e_b,
            dimension_numbers=(((1,), (1,)), ((), ())),
            preferred_element_type=jnp.float32))
    bl = jnp.concatenate(rows, axis=0)
    dist = jnp.exp(ds * (ls_ref[:].astype(jnp.float32) - ts))
    logits = bl + dist
    bb, m = logits.shape
    iota = lax.broadcasted_iota(jnp.int32, (bb, m), 1)
    mask = iota < ne_ref[:]
    logits_ref[:] = jnp.where(mask, logits, jnp.float32(1e-34))
    mask_ref[:] = mask.astype(jnp.int32)


def _h2_body(h_ref, w_ref, out_ref):
    out_ref[:] = jnp.dot(h_ref[:], w_ref[:], preferred_element_type=jnp.float32)


def _sc_kernel(emb_hbm, h2_hbm, ls_hbm, ne_hbm, cst_hbm, out_hbm, mask_hbm,
               ebuf0, ebuf1, h2v, lsv, nev, cstv, orow, mrow, tbuf, sem0, sem1):
    wid = lax.axis_index("s") * _NC + lax.axis_index("c")
    base = wid * _BPW
    emb_base = _B_TC + base
    pltpu.sync_copy(h2_hbm.at[pl.ds(base * _D, _BPW * _D)], h2v)
    pltpu.sync_copy(ls_hbm.at[pl.ds(base * _M, _BPW * _M)], lsv)
    pltpu.sync_copy(ne_hbm, nev)
    pltpu.sync_copy(cst_hbm, cstv)
    iota = lax.broadcasted_iota(jnp.int32, (16,), 0)
    dsv = cstv[pl.ds(0, 16)]
    tsv = cstv[pl.ds(16, 16)]
    sems = [sem0, sem1]
    handles = [None, None]
    ebufs = [ebuf0, ebuf1]
    handles[0] = pltpu.async_copy(emb_hbm.at[emb_base], ebuf0, sem0)
    for bl in range(_BPW):
        buf = bl % 2
        handles[buf].wait()
        if bl + 1 < _BPW:
            nbuf = (bl + 1) % 2
            handles[nbuf] = pltpu.async_copy(
                emb_hbm.at[emb_base + bl + 1], ebufs[nbuf], sems[nbuf])
        ebuf = ebufs[buf]
        h2k = [h2v[pl.ds(bl * _D + k * 16, 16)] for k in range(8)]
        nesp = plsc.load_gather(nev, [jnp.full((16,), base + bl, jnp.int32)])

        def group_body(mg, carry):
            accs = []
            for j in range(16):
                mrow_i = mg * 16 + j
                p = [ebuf[mrow_i, pl.ds(k * 16, 16)] * h2k[k]
                     for k in range(8)]
                q = [p[0] + p[1], p[2] + p[3], p[4] + p[5], p[6] + p[7]]
                accs.append((q[0] + q[1]) + (q[2] + q[3]))
            for j in range(16):
                tbuf[pl.ds(j * 16, 16)] = accs[j]
            cols = [plsc.load_gather(tbuf, [iota * 16 + c]) for c in range(16)]
            while len(cols) > 1:
                cols = [cols[i] + cols[i + 1] for i in range(0, len(cols), 2)]
            tot = cols[0]
            lsx = lsv[pl.ds(bl * _M + mg * 16, 16)]
            dist = jnp.exp(dsv * (lsx.astype(jnp.float32) - tsv))
            logit = tot + dist
            mvec = iota + mg * 16
            mask = mvec < nesp
            orow[pl.ds(mg * 16, 16)] = jnp.where(mask, logit, jnp.float32(1e-34))
            mrow[pl.ds(mg * 16, 16)] = mask.astype(jnp.int32)
            return carry

        lax.fori_loop(0, _M // 16, group_body, 0)
        pltpu.sync_copy(orow, out_hbm.at[base + bl])
        pltpu.sync_copy(mrow, mask_hbm.at[base + bl])


def _run_sc(emb_sc, h2_sc, ls_sc, ne_sc, consts, interpret=False):
    mesh = plsc.VectorSubcoreMesh(core_axis_name="c", subcore_axis_name="s")
    fn = functools.partial(
        pl.kernel, mesh=mesh,
        out_type=[jax.ShapeDtypeStruct((_B_SC, _M), jnp.float32),
                  jax.ShapeDtypeStruct((_B_SC, _M), jnp.int32)],
        scratch_types=[
            pltpu.VMEM((_M, _D), jnp.float32),
            pltpu.VMEM((_M, _D), jnp.float32),
            pltpu.VMEM((_BPW * _D,), jnp.float32),
            pltpu.VMEM((_BPW * _M,), jnp.int32),
            pltpu.VMEM((_B_SC,), jnp.int32),
            pltpu.VMEM((32,), jnp.float32),
            pltpu.VMEM((_M,), jnp.float32),
            pltpu.VMEM((_M,), jnp.int32),
            pltpu.VMEM((_M,), jnp.float32),
            pltpu.SemaphoreType.DMA,
            pltpu.SemaphoreType.DMA,
        ],
        compiler_params=pltpu.CompilerParams(needs_layout_passes=False,
                                             use_tc_tiling_on_sc=True),
        interpret=interpret)(_sc_kernel)
    return fn(emb_sc, h2_sc, ls_sc, ne_sc, consts)


@functools.partial(jax.jit, static_argnames=("interpret",))
def _run(hidden, embeddings, W_proj, distance_scalar, last_seen,
         num_embeddings, timestep, interpret=False):
    B, M, D = embeddings.shape
    Bb = 64
    ds = distance_scalar.astype(jnp.float32)
    ts = jnp.asarray(timestep, jnp.float32)
    scal = jnp.stack([ds, ts]).reshape(1, 2)
    ne = num_embeddings.astype(jnp.int32)
    ls = last_seen.astype(jnp.int32)

    h2_sc = pl.pallas_call(
        _h2_body,
        out_shape=jax.ShapeDtypeStruct((_B_SC, D), jnp.float32),
        interpret=interpret,
    )(hidden[_B_TC:], W_proj)
    consts = jnp.concatenate([jnp.full((16,), ds, jnp.float32),
                              jnp.full((16,), ts, jnp.float32)])
    sc_logits, sc_mask = _run_sc(
        embeddings,
        h2_sc.reshape(_B_SC * _D),
        ls[_B_TC:].reshape(_B_SC * _M),
        ne[_B_TC:],
        consts, interpret=interpret)

    grid = (_B_TC // Bb,)
    tc_logits, tc_mask = pl.pallas_call(
        _tc_body,
        grid=grid,
        in_specs=[
            pl.BlockSpec((1, 2), lambda i: (0, 0)),
            pl.BlockSpec((Bb, D), lambda i: (i, 0)),
            pl.BlockSpec((D, D), lambda i: (0, 0)),
            pl.BlockSpec((Bb, M, D), lambda i: (i, 0, 0)),
            pl.BlockSpec((Bb, M), lambda i: (i, 0)),
            pl.BlockSpec((Bb, 1), lambda i: (i, 0)),
        ],
        out_specs=[
            pl.BlockSpec((Bb, M), lambda i: (i, 0)),
            pl.BlockSpec((Bb, M), lambda i: (i, 0)),
        ],
        out_shape=[
            jax.ShapeDtypeStruct((_B_TC, M), jnp.float32),
            jax.ShapeDtypeStruct((_B_TC, M), jnp.int32),
        ],
        compiler_params=pltpu.CompilerParams(
            dimension_semantics=("parallel",),
            vmem_limit_bytes=100 * 1024 * 1024),
        interpret=interpret,
    )(scal, hidden[:_B_TC], W_proj, embeddings[:_B_TC], ls[:_B_TC],
      ne[:_B_TC].reshape(_B_TC, 1))

    logits = jnp.concatenate([tc_logits, sc_logits], axis=0)
    mask = jnp.concatenate([tc_mask, sc_mask], axis=0).astype(jnp.bool_)
    return logits, mask


def kernel(hidden, embeddings, W_proj, distance_scalar, last_seen,
           num_embeddings, timestep):
    return _run(hidden, embeddings, W_proj, distance_scalar, last_seen,
                num_embeddings, timestep)


# speedup vs baseline: 4.5525x; 2.7617x over previous
import functools

import jax
import jax.numpy as jnp
from jax import lax
from jax.experimental import pallas as pl
from jax.experimental.pallas import tpu as pltpu


def _body(scal_ref, h_ref, w_ref, emb_ref, ls_ref, ne_ref, logits_ref, mask_ref):
    ds = scal_ref[0, 0]
    ts = scal_ref[0, 1]
    h2 = jnp.dot(h_ref[:], w_ref[:], preferred_element_type=jnp.float32)
    h2b = h2.astype(jnp.bfloat16)
    rows = []
    for b in range(h_ref.shape[0]):
        e_b = emb_ref[b].astype(jnp.bfloat16)
        rows.append(lax.dot_general(
            h2b[b:b + 1, :], e_b,
            dimension_numbers=(((1,), (1,)), ((), ())),
            preferred_element_type=jnp.float32))
    bl = jnp.concatenate(rows, axis=0)
    dist = jnp.exp(ds * (ls_ref[:].astype(jnp.float32) - ts))
    logits = bl + dist
    bb, m = logits.shape
    iota = lax.broadcasted_iota(jnp.int32, (bb, m), 1)
    mask = iota < ne_ref[:]
    logits_ref[:] = jnp.where(mask, logits, jnp.float32(1e-34))
    mask_ref[:] = mask


@functools.partial(jax.jit, static_argnames=("interpret",))
def _run(hidden, embeddings, W_proj, distance_scalar, last_seen,
         num_embeddings, timestep, interpret=False):
    B, M, D = embeddings.shape
    Bb = 64
    scal = jnp.stack([distance_scalar.astype(jnp.float32),
                      jnp.asarray(timestep, jnp.float32)]).reshape(1, 2)
    ne2 = num_embeddings.astype(jnp.int32).reshape(B, 1)
    grid = (B // Bb,)
    logits, mask_i = pl.pallas_call(
        _body,
        grid=grid,
        in_specs=[
            pl.BlockSpec((1, 2), lambda i: (0, 0)),
            pl.BlockSpec((Bb, D), lambda i: (i, 0)),
            pl.BlockSpec((D, D), lambda i: (0, 0)),
            pl.BlockSpec((Bb, M, D), lambda i: (i, 0, 0)),
            pl.BlockSpec((Bb, M), lambda i: (i, 0)),
            pl.BlockSpec((Bb, 1), lambda i: (i, 0)),
        ],
        out_specs=[
            pl.BlockSpec((Bb, M), lambda i: (i, 0)),
            pl.BlockSpec((Bb, M), lambda i: (i, 0)),
        ],
        out_shape=[
            jax.ShapeDtypeStruct((B, M), jnp.float32),
            jax.ShapeDtypeStruct((B, M), jnp.bool_),
        ],
        compiler_params=pltpu.CompilerParams(
            dimension_semantics=("parallel",),
            vmem_limit_bytes=100 * 1024 * 1024),
        interpret=interpret,
    )(scal, hidden, W_proj, embeddings, last_seen.astype(jnp.int32), ne2)
    return logits, mask_i


def kernel(hidden, embeddings, W_proj, distance_scalar, last_seen,
           num_embeddings, timestep):
    return _run(hidden, embeddings, W_proj, distance_scalar, last_seen,
                num_embeddings, timestep)
